# baseline (speedup 1.0000x reference)
import functools

import jax
import jax.numpy as jnp
from jax import lax
from jax.experimental import pallas as pl
from jax.experimental.pallas import tpu as pltpu
from jax.experimental.pallas import tpu_sc as plsc

B = 8
N = 128
BN = B * N
E = 32768
EB = E // B
ED = 32
OUT = 64
HEADS = 4
POS = 16
ISQ2 = 0.7071067811865476

_F = jnp.float32


def _silu(x):
    return x / (1.0 + jnp.exp(-x))


def _dot(a, b):
    return jnp.dot(a, b, preferred_element_type=_F)


def _gat(ohT, tab):
    return lax.dot_general(ohT, tab, (((0,), (0,)), ((), ())),
                           preferred_element_type=_F)


def _sc_gather(ori2, spd2, idx4):
    info = plsc.get_sparse_core_info()
    nw = info.num_cores * info.num_subcores
    bpw = E // nw
    ch = bpw // 128
    idx3 = idx4.reshape(nw, ch, 128)
    mesh = plsc.VectorSubcoreMesh(core_axis_name="c", subcore_axis_name="s")

    @functools.partial(
        pl.kernel,
        mesh=mesh,
        out_type=(jax.ShapeDtypeStruct((E, 128), _F),
                  jax.ShapeDtypeStruct((E, 128), _F)),
        scratch_types=[
            pltpu.VMEM((ch, 128), jnp.int32),
            pltpu.VMEM((2, 128, 128), _F),
            pltpu.VMEM((2, 128, 128), _F),
            pltpu.SemaphoreType.DMA,
        ],
    )
    def gather(ori_hbm, spd_hbm, idx_hbm, go_hbm, gs_hbm, idx_v, r1, r2, sem):
        wid = lax.axis_index("s") * info.num_cores + lax.axis_index("c")
        base = wid * bpw
        pltpu.sync_copy(idx_hbm.at[wid], idx_v)

        def drain(j, cps):
            c1, c2 = cps
            c1.wait()
            c2.wait()
            pltpu.sync_copy(r1.at[j % 2],
                            go_hbm.at[pl.ds(base + j * 128, 128)])
            pltpu.sync_copy(r2.at[j % 2],
                            gs_hbm.at[pl.ds(base + j * 128, 128)])

        pend = None
        for j in range(ch):
            cur = (pltpu.async_copy(ori_hbm.at[idx_v.at[j]], r1.at[j % 2],
                                    sem),
                   pltpu.async_copy(spd_hbm.at[idx_v.at[j]], r2.at[j % 2],
                                    sem))
            if pend is not None:
                drain(j - 1, pend)
            pend = cur
        drain(ch - 1, pend)

    return gather(ori2, spd2, idx3)


def _conv_body(xd_ref, xp_ref, go_ref, gs_ref, src_ref, dst_ref, *rest):
    wrefs = rest[:33]
    qouts = rest[33:]

    h = xd_ref[0]
    hp = xp_ref[0]
    src = src_ref[0]
    dst = dst_ref[0]

    iota_n = lax.broadcasted_iota(jnp.int32, (N, EB), 0)
    ohs = (jnp.broadcast_to(src, (N, EB)) == iota_n).astype(_F)
    ohd = (jnp.broadcast_to(dst, (N, EB)) == iota_n).astype(_F)
    ohsum = ohs + ohd
    cnt = jnp.maximum(jnp.sum(ohd, axis=1, keepdims=True), 1.0)

    par_tab = jnp.float32(1.0) * (lax.broadcasted_iota(
        jnp.int32, (N, 1), 0) % 4).astype(_F)
    par = _gat(ohd, par_tab)
    gparts = []
    for wref in (go_ref, gs_ref):
        wide = wref[0]
        sel = jnp.zeros((EB, ED), _F)
        for p in range(4):
            mask = jnp.broadcast_to(par == float(p), (EB, ED))
            sel = jnp.where(mask, wide[:, 32 * p:32 * (p + 1)], sel)
        gparts.append(sel)
    g = jnp.concatenate(gparts, axis=1)

    hm = (lax.broadcasted_iota(jnp.int32, (OUT, HEADS), 0) // 16
          == lax.broadcasted_iota(jnp.int32, (OUT, HEADS), 1)).astype(_F)
    hmT = (lax.broadcasted_iota(jnp.int32, (HEADS, OUT), 0)
           == lax.broadcasted_iota(jnp.int32, (HEADS, OUT), 1) // 16).astype(_F)

    qtabs = []
    for i in range(3):
        (wq, bq, wk, bk, wv, bv, we, be, wpos, wec, bec) = (
            r[...] for r in wrefs[11 * i:11 * (i + 1)])
        qn = _dot(h, wq) + bq
        kn = _dot(h, wk) + bk
        vn = _dot(h, wv) + bv
        pmn = _dot(hp, wpos)

        gq = _gat(ohd, qn)
        gk = _gat(ohs, kn)
        gv = _gat(ohs, vn)
        gpm = _gat(ohs, pmn)

        if i == 0:
            he = g
        else:
            he = g * (ISQ2 ** i)
            for j in range(i):
                becj = wrefs[11 * j + 10][...]
                qsum = _gat(ohsum, qtabs[j])
                he = he + (ISQ2 ** (i - j)) * _silu(qsum + becj)

        ek = _dot(he, we) + be
        k_e = gk + ek
        v_e = gv + ek
        logits = _dot(gq * k_e, hm) * 0.25

        s = _dot(ohd, logits) / cnt
        ex = jnp.exp(logits - _gat(ohd, s))
        den = _dot(ohd, ex)
        alpha = ex / (_gat(ohd, den) + 1e-16)

        aexp = _dot(alpha, hmT)
        h = _dot(ohd, aexp * v_e)
        posw = (jnp.sum(alpha, axis=1, keepdims=True) * 0.25) * gpm
        hp = jnp.tanh(hp + _dot(ohd, posw))

        qi = _dot(h, wec)
        qtabs.append(qi)
        qouts[i][0] = qi


_RT = 32


def _final_body(ori_ref, spd_ref, q0r, q0c, q1r, q1c, q2r, q2c,
                wel_ref, bel_ref, b0_ref, b1_ref, b2_ref, out_ref):
    wel = wel_ref[...]
    w0 = wel[0:OUT] + wel[OUT:2 * OUT] * (ISQ2 * 0.5)
    foT = jnp.transpose(ori_ref[0].reshape(_RT * N, ED))
    fsT = jnp.transpose(spd_ref[0].reshape(_RT * N, ED))
    accT = (_gat(w0[0:ED], foT) + _gat(w0[ED:2 * ED], fsT)
            + jnp.transpose(bel_ref[...]))

    scales = (ISQ2 * 0.5, 0.5, ISQ2)
    for (qr, qc, br, sc) in ((q0r, q0c, b0_ref, scales[0]),
                             (q1r, q1c, b1_ref, scales[1]),
                             (q2r, q2c, b2_ref, scales[2])):
        qrT = jnp.transpose(qr[0])
        qcT = jnp.transpose(qc[0])
        qrow = jnp.broadcast_to(qrT[:, :, None], (OUT, _RT, N))
        qcol = jnp.broadcast_to(qcT[:, None, :], (OUT, _RT, N))
        sT = _silu((qrow + qcol).reshape(OUT, _RT * N)
                   + jnp.transpose(br[...]))
        accT = accT + _gat(wel[OUT:2 * OUT], sT) * sc

    out_ref[0] = accT.reshape(ED, _RT, N)


def kernel(x_degree, x_pos, edge_index, dense_ori, dense_spd, dense_index,
           params):
    ei = edge_index.astype(jnp.int32)
    di = dense_index.astype(jnp.int32)
    ori2 = dense_ori.reshape(E, 128)
    spd2 = dense_spd.reshape(E, 128)

    g_ori, g_spd = _sc_gather(ori2, spd2, di // 4)

    offs = (jnp.arange(B, dtype=jnp.int32) * N)[:, None]
    srcl = (ei[0].reshape(B, EB) - offs).reshape(B, 1, EB)
    dstl = (ei[1].reshape(B, EB) - offs).reshape(B, 1, EB)

    xd3 = x_degree.reshape(B, N, x_degree.shape[1])
    xp3 = x_pos.reshape(B, N, POS)
    go3 = g_ori.reshape(B, EB, 128)
    gs3 = g_spd.reshape(B, EB, 128)

    wlist = []
    for i in range(3):
        p = params["convs"][i]
        wec, bec = params["edge_convs"][i]
        wlist += [p["Wq"], p["bq"].reshape(1, OUT), p["Wk"],
                  p["bk"].reshape(1, OUT), p["Wv"], p["bv"].reshape(1, OUT),
                  p["We"], p["be"].reshape(1, OUT), p["Wpos"], wec,
                  bec.reshape(1, OUT)]

    def _full(a):
        nd = a.ndim
        return pl.BlockSpec(a.shape, lambda b, _n=nd: (0,) * _n)

    in_specs = [
        pl.BlockSpec((1, N, x_degree.shape[1]), lambda b: (b, 0, 0)),
        pl.BlockSpec((1, N, POS), lambda b: (b, 0, 0)),
        pl.BlockSpec((1, EB, 128), lambda b: (b, 0, 0)),
        pl.BlockSpec((1, EB, 128), lambda b: (b, 0, 0)),
        pl.BlockSpec((1, 1, EB), lambda b: (b, 0, 0)),
        pl.BlockSpec((1, 1, EB), lambda b: (b, 0, 0)),
    ] + [_full(a) for a in wlist]

    q0, q1, q2 = pl.pallas_call(
        _conv_body,
        grid=(B,),
        in_specs=in_specs,
        out_specs=[pl.BlockSpec((1, N, OUT), lambda b: (b, 0, 0))] * 3,
        out_shape=[jax.ShapeDtypeStruct((B, N, OUT), _F)] * 3,
    )(xd3, xp3, go3, gs3, srcl, dstl, *wlist)

    wel, bel = params["edge_layer"]
    becs = [params["edge_convs"][i][1].reshape(1, OUT) for i in range(3)]

    nrt = N // _RT
    qrow_spec = pl.BlockSpec((1, _RT, OUT), lambda b, r: (b, r, 0))
    qcol_spec = pl.BlockSpec((1, N, OUT), lambda b, r: (b, 0, 0))

    def _full2(a):
        nd = a.ndim
        return pl.BlockSpec(a.shape, lambda b, r, _n=nd: (0,) * _n)

    out4 = pl.pallas_call(
        _final_body,
        grid=(B, nrt),
        in_specs=[
            pl.BlockSpec((1, _RT, N, ED), lambda b, r: (b, r, 0, 0)),
            pl.BlockSpec((1, _RT, N, ED), lambda b, r: (b, r, 0, 0)),
            qrow_spec, qcol_spec, qrow_spec, qcol_spec, qrow_spec, qcol_spec,
            _full2(wel), _full2(bel.reshape(1, ED)),
            _full2(becs[0]), _full2(becs[1]), _full2(becs[2]),
        ],
        out_specs=pl.BlockSpec((1, ED, _RT, N), lambda b, r: (b, 0, r, 0)),
        out_shape=jax.ShapeDtypeStruct((B, ED, N, N), _F),
    )(dense_ori, dense_spd, q0, q0, q1, q1, q2, q2,
      wel, bel.reshape(1, ED), *becs)

    return out4


# speedup vs baseline: 1.1367x; 1.1367x over previous
import functools

import jax
import jax.numpy as jnp
from jax import lax
from jax.experimental import pallas as pl
from jax.experimental.pallas import tpu as pltpu
from jax.experimental.pallas import tpu_sc as plsc

B = 8
N = 128
BN = B * N
E = 32768
EB = E // B
ED = 32
OUT = 64
HEADS = 4
POS = 16
ISQ2 = 0.7071067811865476

_F = jnp.float32


def _silu(x):
    return x / (1.0 + jnp.exp(-x))


def _dot(a, b):
    return jnp.dot(a, b, preferred_element_type=_F)


def _gat(ohT, tab):
    return lax.dot_general(ohT, tab, (((0,), (0,)), ((), ())),
                           preferred_element_type=_F)


def _sc_gather(ori2, spd2, idx4):
    info = plsc.get_sparse_core_info()
    nw = info.num_cores * info.num_subcores
    bpw = E // nw
    ch = bpw // 128
    idx3 = idx4.reshape(nw, ch, 128)
    mesh = plsc.VectorSubcoreMesh(core_axis_name="c", subcore_axis_name="s")

    @functools.partial(
        pl.kernel,
        mesh=mesh,
        out_type=(jax.ShapeDtypeStruct((E, 128), _F),
                  jax.ShapeDtypeStruct((E, 128), _F)),
        scratch_types=[
            pltpu.VMEM((ch, 128), jnp.int32),
            pltpu.VMEM((2, 128, 128), _F),
            pltpu.VMEM((2, 128, 128), _F),
            pltpu.SemaphoreType.DMA,
        ],
    )
    def gather(ori_hbm, spd_hbm, idx_hbm, go_hbm, gs_hbm, idx_v, r1, r2, sem):
        wid = lax.axis_index("s") * info.num_cores + lax.axis_index("c")
        base = wid * bpw
        pltpu.sync_copy(idx_hbm.at[wid], idx_v)

        def drain(j, cps):
            c1, c2 = cps
            c1.wait()
            c2.wait()
            pltpu.sync_copy(r1.at[j % 2],
                            go_hbm.at[pl.ds(base + j * 128, 128)])
            pltpu.sync_copy(r2.at[j % 2],
                            gs_hbm.at[pl.ds(base + j * 128, 128)])

        pend = None
        for j in range(ch):
            cur = (pltpu.async_copy(ori_hbm.at[idx_v.at[j]], r1.at[j % 2],
                                    sem),
                   pltpu.async_copy(spd_hbm.at[idx_v.at[j]], r2.at[j % 2],
                                    sem))
            if pend is not None:
                drain(j - 1, pend)
            pend = cur
        drain(ch - 1, pend)

    return gather(ori2, spd2, idx3)


def _conv_body(xd_ref, xp_ref, go_ref, gs_ref, src_ref, dst_ref, *rest):
    wrefs = rest[:33]
    qouts = rest[33:]

    h = xd_ref[0]
    hp = xp_ref[0]
    src = src_ref[0]
    dst = dst_ref[0]

    iota_n = lax.broadcasted_iota(jnp.int32, (N, EB), 0)
    ohs = (jnp.broadcast_to(src, (N, EB)) == iota_n).astype(_F)
    ohd = (jnp.broadcast_to(dst, (N, EB)) == iota_n).astype(_F)
    ohsum = ohs + ohd
    cnt = jnp.maximum(jnp.sum(ohd, axis=1, keepdims=True), 1.0)

    par_tab = jnp.float32(1.0) * (lax.broadcasted_iota(
        jnp.int32, (N, 1), 0) % 4).astype(_F)
    par = _gat(ohd, par_tab)
    gparts = []
    for wref in (go_ref, gs_ref):
        wide = wref[0]
        sel = jnp.zeros((EB, ED), _F)
        for p in range(4):
            mask = jnp.broadcast_to(par == float(p), (EB, ED))
            sel = jnp.where(mask, wide[:, 32 * p:32 * (p + 1)], sel)
        gparts.append(sel)
    g = jnp.concatenate(gparts, axis=1)

    hm = (lax.broadcasted_iota(jnp.int32, (OUT, HEADS), 0) // 16
          == lax.broadcasted_iota(jnp.int32, (OUT, HEADS), 1)).astype(_F)
    hmT = (lax.broadcasted_iota(jnp.int32, (HEADS, OUT), 0)
           == lax.broadcasted_iota(jnp.int32, (HEADS, OUT), 1) // 16).astype(_F)

    qtabs = []
    for i in range(3):
        (wq, bq, wk, bk, wv, bv, we, be, wpos, wec, bec) = (
            r[...] for r in wrefs[11 * i:11 * (i + 1)])
        qn = _dot(h, wq) + bq
        kn = _dot(h, wk) + bk
        vn = _dot(h, wv) + bv
        pmn = _dot(hp, wpos)

        gq = _gat(ohd, qn)
        gk = _gat(ohs, kn)
        gv = _gat(ohs, vn)
        gpm = _gat(ohs, pmn)

        if i == 0:
            he = g
        else:
            he = g * (ISQ2 ** i)
            for j in range(i):
                becj = wrefs[11 * j + 10][...]
                qsum = _gat(ohsum, qtabs[j])
                he = he + (ISQ2 ** (i - j)) * _silu(qsum + becj)

        ek = _dot(he, we) + be
        k_e = gk + ek
        v_e = gv + ek
        logits = _dot(gq * k_e, hm) * 0.25

        s = _dot(ohd, logits) / cnt
        ex = jnp.exp(logits - _gat(ohd, s))
        den = _dot(ohd, ex)
        alpha = ex / (_gat(ohd, den) + 1e-16)

        aexp = _dot(alpha, hmT)
        h = _dot(ohd, aexp * v_e)
        posw = (jnp.sum(alpha, axis=1, keepdims=True) * 0.25) * gpm
        hp = jnp.tanh(hp + _dot(ohd, posw))

        qi = _dot(h, wec)
        qtabs.append(qi)
        qouts[i][0] = qi


_RT = 32


def _final_body(ori_ref, spd_ref, q0r, q0c, q1r, q1c, q2r, q2c,
                wel_ref, bel_ref, b0_ref, b1_ref, b2_ref, out_ref):
    wel = wel_ref[...]
    w0 = wel[0:OUT] + wel[OUT:2 * OUT] * (ISQ2 * 0.5)
    foT = jnp.transpose(ori_ref[0], (1, 0, 2)).reshape(ED, _RT * N)
    fsT = jnp.transpose(spd_ref[0], (1, 0, 2)).reshape(ED, _RT * N)
    accT = (_gat(w0[0:ED], foT) + _gat(w0[ED:2 * ED], fsT)
            + jnp.transpose(bel_ref[...]))

    scales = (ISQ2 * 0.5, 0.5, ISQ2)
    for (qr, qc, br, sc) in ((q0r, q0c, b0_ref, scales[0]),
                             (q1r, q1c, b1_ref, scales[1]),
                             (q2r, q2c, b2_ref, scales[2])):
        qrT = jnp.transpose(qr[0])
        qcT = jnp.transpose(qc[0])
        qrow = jnp.broadcast_to(qrT[:, :, None], (OUT, _RT, N))
        qcol = jnp.broadcast_to(qcT[:, None, :], (OUT, _RT, N))
        sT = _silu((qrow + qcol).reshape(OUT, _RT * N)
                   + jnp.transpose(br[...]))
        accT = accT + _gat(wel[OUT:2 * OUT], sT) * sc

    out_ref[0] = accT.reshape(ED, _RT, N)


def kernel(x_degree, x_pos, edge_index, dense_ori, dense_spd, dense_index,
           params):
    ei = edge_index.astype(jnp.int32)
    di = dense_index.astype(jnp.int32)
    ori2 = dense_ori.reshape(E, 128)
    spd2 = dense_spd.reshape(E, 128)

    g_ori, g_spd = _sc_gather(ori2, spd2, di // 4)

    offs = (jnp.arange(B, dtype=jnp.int32) * N)[:, None]
    srcl = (ei[0].reshape(B, EB) - offs).reshape(B, 1, EB)
    dstl = (ei[1].reshape(B, EB) - offs).reshape(B, 1, EB)

    xd3 = x_degree.reshape(B, N, x_degree.shape[1])
    xp3 = x_pos.reshape(B, N, POS)
    go3 = g_ori.reshape(B, EB, 128)
    gs3 = g_spd.reshape(B, EB, 128)

    wlist = []
    for i in range(3):
        p = params["convs"][i]
        wec, bec = params["edge_convs"][i]
        wlist += [p["Wq"], p["bq"].reshape(1, OUT), p["Wk"],
                  p["bk"].reshape(1, OUT), p["Wv"], p["bv"].reshape(1, OUT),
                  p["We"], p["be"].reshape(1, OUT), p["Wpos"], wec,
                  bec.reshape(1, OUT)]

    def _full(a):
        nd = a.ndim
        return pl.BlockSpec(a.shape, lambda b, _n=nd: (0,) * _n)

    in_specs = [
        pl.BlockSpec((1, N, x_degree.shape[1]), lambda b: (b, 0, 0)),
        pl.BlockSpec((1, N, POS), lambda b: (b, 0, 0)),
        pl.BlockSpec((1, EB, 128), lambda b: (b, 0, 0)),
        pl.BlockSpec((1, EB, 128), lambda b: (b, 0, 0)),
        pl.BlockSpec((1, 1, EB), lambda b: (b, 0, 0)),
        pl.BlockSpec((1, 1, EB), lambda b: (b, 0, 0)),
    ] + [_full(a) for a in wlist]

    q0, q1, q2 = pl.pallas_call(
        _conv_body,
        grid=(B,),
        in_specs=in_specs,
        out_specs=[pl.BlockSpec((1, N, OUT), lambda b: (b, 0, 0))] * 3,
        out_shape=[jax.ShapeDtypeStruct((B, N, OUT), _F)] * 3,
    )(xd3, xp3, go3, gs3, srcl, dstl, *wlist)

    wel, bel = params["edge_layer"]
    becs = [params["edge_convs"][i][1].reshape(1, OUT) for i in range(3)]

    nrt = N // _RT
    qrow_spec = pl.BlockSpec((1, _RT, OUT), lambda b, r: (b, r, 0))
    qcol_spec = pl.BlockSpec((1, N, OUT), lambda b, r: (b, 0, 0))

    def _full2(a):
        nd = a.ndim
        return pl.BlockSpec(a.shape, lambda b, r, _n=nd: (0,) * _n)

    out4 = pl.pallas_call(
        _final_body,
        grid=(B, nrt),
        in_specs=[
            pl.BlockSpec((1, _RT, ED, N), lambda b, r: (b, r, 0, 0)),
            pl.BlockSpec((1, _RT, ED, N), lambda b, r: (b, r, 0, 0)),
            qrow_spec, qcol_spec, qrow_spec, qcol_spec, qrow_spec, qcol_spec,
            _full2(wel), _full2(bel.reshape(1, ED)),
            _full2(becs[0]), _full2(becs[1]), _full2(becs[2]),
        ],
        out_specs=pl.BlockSpec((1, ED, _RT, N), lambda b, r: (b, 0, r, 0)),
        out_shape=jax.ShapeDtypeStruct((B, ED, N, N), _F),
    )(dense_ori.transpose(0, 1, 3, 2), dense_spd.transpose(0, 1, 3, 2),
      q0, q0, q1, q1, q2, q2, wel, bel.reshape(1, ED), *becs)

    return out4
